# Initial kernel scaffold; baseline (speedup 1.0000x reference)
import jax
import jax.numpy as jnp
from jax.experimental import pallas as pl


def kernel(atom_feature, edge_index, bond_feature, distance, b_factor, node2graph, W_msg, b_msg, W_node, b_node, W_out, b_out):
    raise NotImplementedError("write your pallas kernel here")


# speedup vs baseline: 1.8646x; 1.8646x over previous
import functools

import jax
import jax.numpy as jnp
from jax import lax
from jax.experimental import pallas as pl
from jax.experimental.pallas import tpu as pltpu
from jax.experimental.pallas import tpu_sc as plsc

N_NODES = 10000
N_EDGES = 320000
D_NODE = 142
D_HID = 128
NUM_GRAPHS = 25
NUM_ENC = 10

NC = 2
NS = 16
NW = NC * NS
EPW = N_EDGES // NW
K = 80
NB = EPW // K
RSTRIPE = 640
RCHUNK = 80


def _p_body(atom_ref, w_ref, o_ref):
    o_ref[...] = jnp.dot(atom_ref[...], w_ref[...],
                         preferred_element_type=jnp.float32)


def _compute_p(atom_feature, w_top):
    return pl.pallas_call(
        _p_body,
        out_shape=jax.ShapeDtypeStruct((N_NODES, D_HID), jnp.float32),
    )(atom_feature, w_top)


_QB = 5000


def _q_body(bond_ref, dist_ref, sc_ref, wb_ref, ws_ref, wc_ref, bias_ref, o_ref):
    xs = dist_ref[...] * sc_ref[...]
    acc = jnp.dot(bond_ref[...], wb_ref[...], preferred_element_type=jnp.float32)
    acc = acc + jnp.dot(jnp.sin(xs), ws_ref[...], preferred_element_type=jnp.float32)
    acc = acc + jnp.dot(jnp.cos(xs), wc_ref[...], preferred_element_type=jnp.float32)
    o_ref[...] = acc + bias_ref[...]


def _compute_q(bond, dist_col, inv_scales, w_bond, w_sin, w_cos, bias_row):
    n_blocks = N_EDGES // _QB
    return pl.pallas_call(
        _q_body,
        grid=(n_blocks,),
        in_specs=[
            pl.BlockSpec((_QB, 5), lambda i: (i, 0)),
            pl.BlockSpec((_QB, 1), lambda i: (i, 0)),
            pl.BlockSpec((1, NUM_ENC), lambda i: (0, 0)),
            pl.BlockSpec((5, D_HID), lambda i: (0, 0)),
            pl.BlockSpec((NUM_ENC, D_HID), lambda i: (0, 0)),
            pl.BlockSpec((NUM_ENC, D_HID), lambda i: (0, 0)),
            pl.BlockSpec((1, D_HID), lambda i: (0, 0)),
        ],
        out_specs=pl.BlockSpec((_QB, D_HID), lambda i: (i, 0)),
        out_shape=jax.ShapeDtypeStruct((N_EDGES, D_HID), jnp.float32),
    )(bond, dist_col, inv_scales, w_bond, w_sin, w_cos, bias_row)


def _sc_agg_body(p_hbm, q_hbm, src_hbm, dst_hbm, out_hbm,
                 src_v, dst_v, q_v, rows_v, stg_v, agg_sh, sem):
    c = lax.axis_index("c")
    s = lax.axis_index("s")
    wid = c * NS + s
    base_r = s * RSTRIPE
    n_chunks = jnp.where(s == NS - 1, 5, 8)

    def _zero(j, _):
        stg_v[j // 8, pl.ds((j % 8) * 16, 16)] = jnp.zeros((16,), jnp.float32)
        return 0
    lax.fori_loop(0, RCHUNK * 8, _zero, 0)

    def _zinit(i, _):
        pltpu.sync_copy(stg_v, agg_sh.at[pl.ds(base_r + i * RCHUNK, RCHUNK)])
        return 0
    lax.fori_loop(0, n_chunks, _zinit, 0)
    plsc.subcore_barrier()

    def _block(b, _):
        e0 = wid * EPW + b * K
        pltpu.sync_copy(src_hbm.at[pl.ds(e0, K)], src_v)
        pltpu.sync_copy(dst_hbm.at[pl.ds(e0, K)], dst_v)
        pltpu.sync_copy(q_hbm.at[pl.ds(e0, K)], q_v)
        pltpu.async_copy(p_hbm.at[src_v], rows_v, sem).wait()

        def _relu_row(e, _):
            for kk in range(D_HID // 16):
                sl = pl.ds(kk * 16, 16)
                rows_v[e, sl] = jnp.maximum(rows_v[e, sl] + q_v[e, sl], 0.0)
            return 0
        lax.fori_loop(0, K, _relu_row, 0)

        pltpu.sync_copy(rows_v, agg_sh.at[dst_v], add=True)
        return 0
    lax.fori_loop(0, NB, _block, 0)

    plsc.subcore_barrier()

    def _out(i, _):
        r0 = base_r + i * RCHUNK
        pltpu.sync_copy(agg_sh.at[pl.ds(r0, RCHUNK)], stg_v)
        pltpu.sync_copy(stg_v, out_hbm.at[c, pl.ds(r0, RCHUNK)])
        return 0
    lax.fori_loop(0, n_chunks, _out, 0)


@functools.cache
def _get_sc_agg():
  return functools.partial(
    pl.kernel,
    out_type=jax.ShapeDtypeStruct((NC, N_NODES, D_HID), jnp.float32),
    mesh=plsc.VectorSubcoreMesh(core_axis_name="c", subcore_axis_name="s",
                                num_cores=NC, num_subcores=NS),
    scratch_types=[
        pltpu.VMEM((K,), jnp.int32),
        pltpu.VMEM((K,), jnp.int32),
        pltpu.VMEM((K, D_HID), jnp.float32),
        pltpu.VMEM((K, D_HID), jnp.float32),
        pltpu.VMEM((RCHUNK, D_HID), jnp.float32),
        pltpu.VMEM_SHARED((N_NODES, D_HID), jnp.float32),
        pltpu.SemaphoreType.DMA,
    ],
  )(_sc_agg_body)


def _final_body(atom_ref, agg_ref, n2g_ref, wnt_ref, wnb_ref, bn_ref,
                wo_ref, bo_ref, o_ref):
    agg = agg_ref[0] + agg_ref[1]
    h = jnp.dot(atom_ref[...], wnt_ref[...], preferred_element_type=jnp.float32)
    h = h + jnp.dot(agg, wnb_ref[...], preferred_element_type=jnp.float32)
    h = jnp.maximum(h + bn_ref[...], 0.0)
    gids = lax.broadcasted_iota(jnp.int32, (N_NODES, D_HID), 1)
    mask = (n2g_ref[...] == gids).astype(jnp.float32)
    dn = (((0,), (0,)), ((), ()))
    gsum = lax.dot_general(mask, h, dn, preferred_element_type=jnp.float32)
    ones = jnp.ones((N_NODES, 1), jnp.float32)
    gcnt = lax.dot_general(mask, ones, dn, preferred_element_type=jnp.float32)
    gmean = gsum / jnp.maximum(gcnt, 1.0)
    h = h + jnp.dot(mask, gmean, preferred_element_type=jnp.float32)
    logits = jnp.dot(h, wo_ref[...], preferred_element_type=jnp.float32) + bo_ref[...]
    o_ref[...] = 1.0 / (1.0 + jnp.exp(-logits))


def _compute_out(atom_feature, agg2, n2g_col, wn_top, wn_bot, bn_row, w_out, bo_row):
    return pl.pallas_call(
        _final_body,
        out_shape=jax.ShapeDtypeStruct((N_NODES, 1), jnp.float32),
    )(atom_feature, agg2, n2g_col, wn_top, wn_bot, bn_row, w_out, bo_row)


def kernel(atom_feature, edge_index, bond_feature, distance, b_factor, node2graph,
           W_msg, b_msg, W_node, b_node, W_out, b_out):
    w_atom = W_msg[:D_NODE]
    w_bond = W_msg[D_NODE:D_NODE + 5]
    w_sin = W_msg[D_NODE + 5:D_NODE + 5 + NUM_ENC]
    w_cos = W_msg[D_NODE + 5 + NUM_ENC:]
    inv_scales = (1.0 / (2.0 ** jnp.arange(NUM_ENC, dtype=jnp.float32)))[None, :]

    p = _compute_p(atom_feature, w_atom)
    q = _compute_q(bond_feature, distance[:, None], inv_scales,
                   w_bond, w_sin, w_cos, b_msg[None, :])
    agg2 = _get_sc_agg()(p, q, edge_index[0], edge_index[1])
    out2 = _compute_out(atom_feature, agg2, node2graph[:, None],
                        W_node[:D_NODE], W_node[D_NODE:], b_node[None, :],
                        W_out, b_out[None, :])
    return (out2[:, 0], b_factor)
